# baseline (speedup 1.0000x reference)
import functools
import math

import jax
import jax.numpy as jnp
import numpy as np
from jax import lax
from jax.experimental import pallas as pl
from jax.experimental.pallas import tpu as pltpu
from jax.experimental.pallas import tpu_sc as plsc

_THRESH = 0.7
_N_MIN = 131072
_CT_F32 = np.float32(-math.log(_THRESH))
_CT_KEY = int(np.array(_CT_F32, np.float32).view(np.int32))
_KEY_HI = 0x7F800000

_SC_NC = 2
_SC_NS = 16
_SC_NW = _SC_NC * _SC_NS
_SC_CHUNK_ROWS = 16


def _nll_body(lg_ref, lab_ref, nll_ref):
    lg = lg_ref[0]
    lab = lab_ref[0]
    m = jnp.max(lg, axis=0)
    s = jnp.sum(jnp.exp(lg - m[None]), axis=0)
    cidx = lax.broadcasted_iota(jnp.int32, lg.shape, 0)
    x = jnp.sum(jnp.where(cidx == lab[None], lg, 0.0), axis=0)
    nll_ref[...] = m + jnp.log(s) - x


def _sc_part_body(rows_per_w, nll_hbm, c0_hbm, s0_hbm,
                  buf0, buf1, accc, accs, sem0, sem1):
    wid = lax.axis_index("s") * _SC_NC + lax.axis_index("c")
    base = wid * rows_per_w
    nchunk = rows_per_w // _SC_CHUNK_ROWS
    bufs = (buf0, buf1)
    sems = (sem0, sem1)

    def start(ci):
        src = nll_hbm.at[pl.ds(base + ci * _SC_CHUNK_ROWS, _SC_CHUNK_ROWS)]
        return pltpu.async_copy(src, bufs[ci % 2], sems[ci % 2])

    zero = jnp.zeros((16,), jnp.float32)
    acc = (zero,) * 8
    pending = start(0)
    for ci in range(nchunk):
        pending.wait()
        if ci + 1 < nchunk:
            pending = start(ci + 1)
        buf = bufs[ci % 2]

        def row_step(r, a, buf=buf):
            a = list(a)
            for k in range(32):
                v = buf[r, pl.ds(k * 16, 16)]
                msk = v >= _CT_F32
                a[k % 4] = a[k % 4] + jnp.where(msk, 1.0, 0.0)
                a[4 + k % 4] = a[4 + k % 4] + jnp.where(msk, v, 0.0)
            return tuple(a)

        acc = lax.fori_loop(0, _SC_CHUNK_ROWS, row_step, acc)

    accc[0] = acc[0] + acc[1] + acc[2] + acc[3]
    accs[0] = acc[4] + acc[5] + acc[6] + acc[7]
    pltpu.sync_copy(accc, c0_hbm.at[pl.ds(wid, 1)])
    pltpu.sync_copy(accs, s0_hbm.at[pl.ds(wid, 1)])


def _select_body(n_pix, nll_a_ref, nll_b_ref, out_ref):
    nll_a = nll_a_ref[...]
    nll_b = nll_b_ref[...]
    keys_a = lax.bitcast_convert_type(nll_a, jnp.int32)
    keys_b = lax.bitcast_convert_type(nll_b, jnp.int32)
    target = jnp.int32(n_pix - _N_MIN)

    def step(_, carry):
        lo, hi = carry
        mid = lo + (hi - lo) // 2
        cnt = (jnp.sum((keys_a <= mid).astype(jnp.int32))
               + jnp.sum((keys_b <= mid).astype(jnp.int32)))
        return (jnp.where(cnt >= target, lo, mid + 1),
                jnp.where(cnt >= target, mid, hi))

    lo, hi = lax.fori_loop(0, 31, step, (jnp.int32(0), jnp.int32(_KEY_HI)))
    t_key = jnp.minimum(lo, jnp.int32(_CT_KEY))
    valid_a = keys_a >= t_key
    valid_b = keys_b >= t_key
    cnt_v = (jnp.sum(valid_a.astype(jnp.int32))
             + jnp.sum(valid_b.astype(jnp.int32)))
    s = (jnp.sum(jnp.where(valid_a, nll_a, 0.0))
         + jnp.sum(jnp.where(valid_b, nll_b, 0.0)))
    out_ref[0, 0] = s / jnp.maximum(cnt_v.astype(jnp.float32), 1.0)


def _nll_call(logits, labels, bh, n0, n1):
    _, c, h, w = logits.shape
    bpn = h // bh
    n_rows = (n1 - n0) * h * w // 512
    return pl.pallas_call(
        _nll_body,
        grid=(n1 - n0, bpn),
        in_specs=[
            pl.BlockSpec((1, c, bh, w), lambda i, j, n0=n0: (i + n0, 0, j, 0)),
            pl.BlockSpec((1, bh, w), lambda i, j, n0=n0: (i + n0, j, 0)),
        ],
        out_specs=pl.BlockSpec((bh, w), lambda i, j, bpn=bpn: (i * bpn + j, 0)),
        out_shape=jax.ShapeDtypeStruct((n_rows, 512), jnp.float32),
    )(logits, labels)


def _sc_call(nll):
    n_rows = nll.shape[0]
    rows_per_w = n_rows // _SC_NW
    mesh = plsc.VectorSubcoreMesh(core_axis_name="c", subcore_axis_name="s")
    return pl.kernel(
        functools.partial(_sc_part_body, rows_per_w),
        out_type=[
            jax.ShapeDtypeStruct((_SC_NW, 16), jnp.float32),
            jax.ShapeDtypeStruct((_SC_NW, 16), jnp.float32),
        ],
        mesh=mesh,
        scratch_types=[
            pltpu.VMEM((_SC_CHUNK_ROWS, 512), jnp.float32),
            pltpu.VMEM((_SC_CHUNK_ROWS, 512), jnp.float32),
            pltpu.VMEM((1, 16), jnp.float32),
            pltpu.VMEM((1, 16), jnp.float32),
            pltpu.SemaphoreType.DMA,
            pltpu.SemaphoreType.DMA,
        ],
    )(nll)


def kernel(logits, labels):
    n, c, h, w = logits.shape
    bh = 64
    n_pix = n * h * w
    n_a = n - 1
    nll_a = _nll_call(logits, labels, bh, 0, n_a)
    c0pa, s0pa = _sc_call(nll_a)
    nll_b = _nll_call(logits, labels, bh, n_a, n)
    c0pb, s0pb = _sc_call(nll_b)
    c0v = jnp.sum(c0pa) + jnp.sum(c0pb)
    s0v = jnp.sum(s0pa) + jnp.sum(s0pb)

    def _slow(_):
        loss = pl.pallas_call(
            functools.partial(_select_body, n_pix),
            out_specs=pl.BlockSpec(memory_space=pltpu.SMEM),
            out_shape=jax.ShapeDtypeStruct((1, 1), jnp.float32),
        )(nll_a, nll_b)
        return loss[0, 0]

    return lax.cond(c0v > _N_MIN, lambda _: s0v / c0v, _slow, operand=None)


# speedup vs baseline: 1.0810x; 1.0810x over previous
import functools
import math

import jax
import jax.numpy as jnp
import numpy as np
from jax import lax
from jax.experimental import pallas as pl
from jax.experimental.pallas import tpu as pltpu
from jax.experimental.pallas import tpu_sc as plsc

_THRESH = 0.7
_N_MIN = 131072
_CT_F32 = np.float32(-math.log(_THRESH))
_CT_KEY = int(np.array(_CT_F32, np.float32).view(np.int32))
_KEY_HI = 0x7F800000

_SC_NC = 2
_SC_NS = 16
_SC_NW = _SC_NC * _SC_NS
_SC_CHUNK_ROWS = 32


def _nll_body(lg_ref, lab_ref, nll_ref):
    lg = lg_ref[0]
    lab = lab_ref[0]
    m = jnp.max(lg, axis=0)
    s = jnp.sum(jnp.exp(lg - m[None]), axis=0)
    cidx = lax.broadcasted_iota(jnp.int32, lg.shape, 0)
    x = jnp.sum(jnp.where(cidx == lab[None], lg, 0.0), axis=0)
    nll_ref[...] = m + jnp.log(s) - x


def _sc_part_body(rows_per_w, nll_hbm, c0_hbm, s0_hbm,
                  buf0, buf1, accc, accs, sem0, sem1):
    wid = lax.axis_index("s") * _SC_NC + lax.axis_index("c")
    base = wid * rows_per_w
    nchunk = rows_per_w // _SC_CHUNK_ROWS
    bufs = (buf0, buf1)
    sems = (sem0, sem1)

    def start(ci):
        src = nll_hbm.at[pl.ds(base + ci * _SC_CHUNK_ROWS, _SC_CHUNK_ROWS)]
        return pltpu.async_copy(src, bufs[ci % 2], sems[ci % 2])

    zf = jnp.zeros((16,), jnp.float32)
    acc = (zf,) * 8
    pending = start(0)
    for ci in range(nchunk):
        pending.wait()
        if ci + 1 < nchunk:
            pending = start(ci + 1)
        buf = bufs[ci % 2]

        def row_step(r, a, buf=buf):
            a = list(a)
            for k in range(32):
                v = buf[r, pl.ds(k * 16, 16)]
                msk = v >= _CT_F32
                a[k % 4] = a[k % 4] + jnp.where(msk, 1.0, 0.0)
                a[4 + k % 4] = a[4 + k % 4] + jnp.where(msk, v, 0.0)
            return tuple(a)

        acc = lax.fori_loop(0, _SC_CHUNK_ROWS, row_step, acc)

    accc[0] = acc[0] + acc[1] + acc[2] + acc[3]
    accs[0] = acc[4] + acc[5] + acc[6] + acc[7]
    pltpu.sync_copy(accc, c0_hbm.at[pl.ds(wid, 1)])
    pltpu.sync_copy(accs, s0_hbm.at[pl.ds(wid, 1)])


def _stagec_body(n_pix, c0p_ref, s0p_ref, nll_any, out_ref, nll_vmem, sem):
    c0 = jnp.sum(c0p_ref[...])
    s0 = jnp.sum(s0p_ref[...])

    @pl.when(c0 > jnp.float32(_N_MIN))
    def _fast():
        out_ref[0, 0] = s0 / c0

    @pl.when(c0 <= jnp.float32(_N_MIN))
    def _slow():
        pltpu.make_async_copy(nll_any, nll_vmem, sem).start()
        pltpu.make_async_copy(nll_any, nll_vmem, sem).wait()
        nll = nll_vmem[...]
        keys = lax.bitcast_convert_type(nll, jnp.int32)
        target = jnp.int32(n_pix - _N_MIN)

        def step(_, carry):
            lo, hi = carry
            mid = lo + (hi - lo) // 2
            cnt = jnp.sum((keys <= mid).astype(jnp.int32))
            return (jnp.where(cnt >= target, lo, mid + 1),
                    jnp.where(cnt >= target, mid, hi))

        lo, hi = lax.fori_loop(0, 31, step, (jnp.int32(0), jnp.int32(_KEY_HI)))
        t_key = jnp.minimum(lo, jnp.int32(_CT_KEY))
        valid = keys >= t_key
        cnt_v = jnp.sum(valid.astype(jnp.int32))
        s = jnp.sum(jnp.where(valid, nll, 0.0))
        out_ref[0, 0] = s / jnp.maximum(cnt_v.astype(jnp.float32), 1.0)


def kernel(logits, labels):
    n, c, h, w = logits.shape
    bh = 64
    n_pix = n * h * w
    n_rows = n_pix // 512
    bpn = h // bh
    nll = pl.pallas_call(
        _nll_body,
        grid=(n, bpn),
        in_specs=[
            pl.BlockSpec((1, c, bh, w), lambda i, j: (i, 0, j, 0)),
            pl.BlockSpec((1, bh, w), lambda i, j: (i, j, 0)),
        ],
        out_specs=pl.BlockSpec((bh, w), lambda i, j, bpn=bpn: (i * bpn + j, 0)),
        out_shape=jax.ShapeDtypeStruct((n_rows, 512), jnp.float32),
    )(logits, labels)

    rows_per_w = n_rows // _SC_NW
    mesh = plsc.VectorSubcoreMesh(core_axis_name="c", subcore_axis_name="s")
    c0p, s0p = pl.kernel(
        functools.partial(_sc_part_body, rows_per_w),
        out_type=[
            jax.ShapeDtypeStruct((_SC_NW, 16), jnp.float32),
            jax.ShapeDtypeStruct((_SC_NW, 16), jnp.float32),
        ],
        mesh=mesh,
        scratch_types=[
            pltpu.VMEM((_SC_CHUNK_ROWS, 512), jnp.float32),
            pltpu.VMEM((_SC_CHUNK_ROWS, 512), jnp.float32),
            pltpu.VMEM((1, 16), jnp.float32),
            pltpu.VMEM((1, 16), jnp.float32),
            pltpu.SemaphoreType.DMA,
            pltpu.SemaphoreType.DMA,
        ],
    )(nll)

    loss = pl.pallas_call(
        functools.partial(_stagec_body, n_pix),
        in_specs=[
            pl.BlockSpec(memory_space=pltpu.VMEM),
            pl.BlockSpec(memory_space=pltpu.VMEM),
            pl.BlockSpec(memory_space=pl.ANY),
        ],
        out_specs=pl.BlockSpec(memory_space=pltpu.SMEM),
        out_shape=jax.ShapeDtypeStruct((1, 1), jnp.float32),
        scratch_shapes=[
            pltpu.VMEM((n_rows, 512), jnp.float32),
            pltpu.SemaphoreType.DMA,
        ],
    )(c0p, s0p, nll)
    return loss[0, 0]


# speedup vs baseline: 1.2817x; 1.1857x over previous
import functools
import math

import jax
import jax.numpy as jnp
import numpy as np
from jax import lax
from jax.experimental import pallas as pl
from jax.experimental.pallas import tpu as pltpu
from jax.experimental.pallas import tpu_sc as plsc

_THRESH = 0.7
_N_MIN = 131072
_CT_F32 = np.float32(-math.log(_THRESH))
_CT_KEY = int(np.array(_CT_F32, np.float32).view(np.int32))
_KEY_HI = 0x7F800000

_SC_NC = 2
_SC_NS = 16
_SC_NW = _SC_NC * _SC_NS
_SC_CHUNK_ROWS = 32


def _nll_body(lg_ref, lab_ref, nll_ref):
    lg = lg_ref[0]
    lab = lab_ref[0]
    m = jnp.max(lg, axis=0)
    s = jnp.sum(jnp.exp(lg - m[None]), axis=0)
    cidx = lax.broadcasted_iota(jnp.int32, lg.shape, 0)
    x = jnp.sum(jnp.where(cidx == lab[None], lg, 0.0), axis=0)
    nll_ref[...] = m + jnp.log(s) - x


def _sc_part_body(rows_per_w, nll_hbm, c0_hbm, s0_hbm,
                  buf0, buf1, accc, accs, sem0, sem1):
    wid = lax.axis_index("s") * _SC_NC + lax.axis_index("c")
    base = wid * rows_per_w
    nchunk = rows_per_w // _SC_CHUNK_ROWS
    bufs = (buf0, buf1)
    sems = (sem0, sem1)

    def start(ci):
        src = nll_hbm.at[pl.ds(base + ci * _SC_CHUNK_ROWS, _SC_CHUNK_ROWS)]
        return pltpu.async_copy(src, bufs[ci % 2], sems[ci % 2])

    zf = jnp.zeros((16,), jnp.float32)
    acc = (zf,) * 8
    pending = start(0)
    for ci in range(nchunk):
        pending.wait()
        if ci + 1 < nchunk:
            pending = start(ci + 1)
        buf = bufs[ci % 2]

        def row_step(r, a, buf=buf):
            a = list(a)
            for k in range(32):
                v = buf[r, pl.ds(k * 16, 16)]
                msk = v >= _CT_F32
                a[k % 4] = a[k % 4] + jnp.where(msk, 1.0, 0.0)
                a[4 + k % 4] = a[4 + k % 4] + jnp.where(msk, v, 0.0)
            return tuple(a)

        acc = lax.fori_loop(0, _SC_CHUNK_ROWS, row_step, acc)

    accc[0] = acc[0] + acc[1] + acc[2] + acc[3]
    accs[0] = acc[4] + acc[5] + acc[6] + acc[7]
    pltpu.sync_copy(accc, c0_hbm.at[pl.ds(wid, 1)])
    pltpu.sync_copy(accs, s0_hbm.at[pl.ds(wid, 1)])


def _stagec_body(n_pix, c0p_ref, s0p_ref, nll_any, out_ref, nll_vmem, sem):
    c0 = jnp.sum(c0p_ref[...])
    s0 = jnp.sum(s0p_ref[...])

    @pl.when(c0 > jnp.float32(_N_MIN))
    def _fast():
        out_ref[0, 0] = s0 / c0

    @pl.when(c0 <= jnp.float32(_N_MIN))
    def _slow():
        pltpu.make_async_copy(nll_any, nll_vmem, sem).start()
        pltpu.make_async_copy(nll_any, nll_vmem, sem).wait()
        nll = nll_vmem[...]
        keys = lax.bitcast_convert_type(nll, jnp.int32)
        target = jnp.int32(n_pix - _N_MIN)

        def step(_, carry):
            lo, hi = carry
            mid = lo + (hi - lo) // 2
            cnt = jnp.sum((keys <= mid).astype(jnp.int32))
            return (jnp.where(cnt >= target, lo, mid + 1),
                    jnp.where(cnt >= target, mid, hi))

        lo, hi = lax.fori_loop(0, 31, step, (jnp.int32(0), jnp.int32(_KEY_HI)))
        t_key = jnp.minimum(lo, jnp.int32(_CT_KEY))
        valid = keys >= t_key
        cnt_v = jnp.sum(valid.astype(jnp.int32))
        s = jnp.sum(jnp.where(valid, nll, 0.0))
        out_ref[0, 0] = s / jnp.maximum(cnt_v.astype(jnp.float32), 1.0)


def kernel(logits, labels):
    n, c, h, w = logits.shape
    bh = 128
    n_pix = n * h * w
    n_rows = n_pix // 512
    bpn = h // bh
    nll = pl.pallas_call(
        _nll_body,
        grid=(n, bpn),
        in_specs=[
            pl.BlockSpec((1, c, bh, w), lambda i, j: (i, 0, j, 0)),
            pl.BlockSpec((1, bh, w), lambda i, j: (i, j, 0)),
        ],
        out_specs=pl.BlockSpec((bh, w), lambda i, j, bpn=bpn: (i * bpn + j, 0)),
        out_shape=jax.ShapeDtypeStruct((n_rows, 512), jnp.float32),
    )(logits, labels)

    rows_per_w = n_rows // _SC_NW
    mesh = plsc.VectorSubcoreMesh(core_axis_name="c", subcore_axis_name="s")
    c0p, s0p = pl.kernel(
        functools.partial(_sc_part_body, rows_per_w),
        out_type=[
            jax.ShapeDtypeStruct((_SC_NW, 16), jnp.float32),
            jax.ShapeDtypeStruct((_SC_NW, 16), jnp.float32),
        ],
        mesh=mesh,
        scratch_types=[
            pltpu.VMEM((_SC_CHUNK_ROWS, 512), jnp.float32),
            pltpu.VMEM((_SC_CHUNK_ROWS, 512), jnp.float32),
            pltpu.VMEM((1, 16), jnp.float32),
            pltpu.VMEM((1, 16), jnp.float32),
            pltpu.SemaphoreType.DMA,
            pltpu.SemaphoreType.DMA,
        ],
    )(nll)

    loss = pl.pallas_call(
        functools.partial(_stagec_body, n_pix),
        in_specs=[
            pl.BlockSpec(memory_space=pltpu.VMEM),
            pl.BlockSpec(memory_space=pltpu.VMEM),
            pl.BlockSpec(memory_space=pl.ANY),
        ],
        out_specs=pl.BlockSpec(memory_space=pltpu.SMEM),
        out_shape=jax.ShapeDtypeStruct((1, 1), jnp.float32),
        scratch_shapes=[
            pltpu.VMEM((n_rows, 512), jnp.float32),
            pltpu.SemaphoreType.DMA,
        ],
    )(c0p, s0p, nll)
    return loss[0, 0]


# speedup vs baseline: 1.3953x; 1.0886x over previous
import functools
import math

import jax
import jax.numpy as jnp
import numpy as np
from jax import lax
from jax.experimental import pallas as pl
from jax.experimental.pallas import tpu as pltpu
from jax.experimental.pallas import tpu_sc as plsc

_THRESH = 0.7
_N_MIN = 131072
_CT_F32 = np.float32(-math.log(_THRESH))
_CT_KEY = int(np.array(_CT_F32, np.float32).view(np.int32))
_KEY_HI = 0x7F800000

_SC_NC = 2
_SC_NS = 16
_SC_NW = _SC_NC * _SC_NS
_SC_CHUNK_ROWS = 32


def _nll_body(lg_ref, lab_ref, nll_ref):
    lg = lg_ref[0]
    lab = lab_ref[0]
    m = jnp.max(lg, axis=0)
    s = jnp.sum(jnp.exp(lg - m[None]), axis=0)
    cidx = lax.broadcasted_iota(jnp.int32, lg.shape, 0)
    x = jnp.sum(jnp.where(cidx == lab[None], lg, 0.0), axis=0)
    nll_ref[...] = m + jnp.log(s) - x


def _sc_part_body(rows_per_w, nll_hbm, c0_hbm, s0_hbm,
                  buf0, buf1, accc, accs, sem0, sem1):
    wid = lax.axis_index("s") * _SC_NC + lax.axis_index("c")
    base = wid * rows_per_w
    nchunk = rows_per_w // _SC_CHUNK_ROWS
    bufs = (buf0, buf1)
    sems = (sem0, sem1)

    def start(ci):
        src = nll_hbm.at[pl.ds(base + ci * _SC_CHUNK_ROWS, _SC_CHUNK_ROWS)]
        return pltpu.async_copy(src, bufs[ci % 2], sems[ci % 2])

    zf = jnp.zeros((16,), jnp.float32)
    acc = (zf,) * 8
    pending = start(0)
    for ci in range(nchunk):
        pending.wait()
        if ci + 1 < nchunk:
            pending = start(ci + 1)
        buf = bufs[ci % 2]

        def row_step(r, a, buf=buf):
            a = list(a)
            for k in range(32):
                v = buf[r, pl.ds(k * 16, 16)]
                msk = v >= _CT_F32
                a[k % 4] = a[k % 4] + jnp.where(msk, 1.0, 0.0)
                a[4 + k % 4] = a[4 + k % 4] + jnp.where(msk, v, 0.0)
            return tuple(a)

        acc = lax.fori_loop(0, _SC_CHUNK_ROWS, row_step, acc)

    accc[0] = acc[0] + acc[1] + acc[2] + acc[3]
    accs[0] = acc[4] + acc[5] + acc[6] + acc[7]
    pltpu.sync_copy(accc, c0_hbm.at[pl.ds(wid, 1)])
    pltpu.sync_copy(accs, s0_hbm.at[pl.ds(wid, 1)])


def _stagec_body(n_pix, c0p_ref, s0p_ref, nll_any, out_ref, nll_vmem, sem):
    c0 = jnp.sum(c0p_ref[...])
    s0 = jnp.sum(s0p_ref[...])

    @pl.when(c0 > jnp.float32(_N_MIN))
    def _fast():
        out_ref[0, 0] = s0 / c0

    @pl.when(c0 <= jnp.float32(_N_MIN))
    def _slow():
        pltpu.make_async_copy(nll_any, nll_vmem, sem).start()
        pltpu.make_async_copy(nll_any, nll_vmem, sem).wait()
        nll = nll_vmem[...]
        keys = lax.bitcast_convert_type(nll, jnp.int32)
        target = jnp.int32(n_pix - _N_MIN)

        def step(_, carry):
            lo, hi = carry
            mid = lo + (hi - lo) // 2
            cnt = jnp.sum((keys <= mid).astype(jnp.int32))
            return (jnp.where(cnt >= target, lo, mid + 1),
                    jnp.where(cnt >= target, mid, hi))

        lo, hi = lax.fori_loop(0, 31, step, (jnp.int32(0), jnp.int32(_KEY_HI)))
        t_key = jnp.minimum(lo, jnp.int32(_CT_KEY))
        valid = keys >= t_key
        cnt_v = jnp.sum(valid.astype(jnp.int32))
        s = jnp.sum(jnp.where(valid, nll, 0.0))
        out_ref[0, 0] = s / jnp.maximum(cnt_v.astype(jnp.float32), 1.0)


def kernel(logits, labels):
    n, c, h, w = logits.shape
    bh = 256
    n_pix = n * h * w
    n_rows = n_pix // 512
    bpn = h // bh
    nll = pl.pallas_call(
        _nll_body,
        grid=(n, bpn),
        in_specs=[
            pl.BlockSpec((1, c, bh, w), lambda i, j: (i, 0, j, 0)),
            pl.BlockSpec((1, bh, w), lambda i, j: (i, j, 0)),
        ],
        out_specs=pl.BlockSpec((bh, w), lambda i, j, bpn=bpn: (i * bpn + j, 0)),
        out_shape=jax.ShapeDtypeStruct((n_rows, 512), jnp.float32),
    )(logits, labels)

    rows_per_w = n_rows // _SC_NW
    mesh = plsc.VectorSubcoreMesh(core_axis_name="c", subcore_axis_name="s")
    c0p, s0p = pl.kernel(
        functools.partial(_sc_part_body, rows_per_w),
        out_type=[
            jax.ShapeDtypeStruct((_SC_NW, 16), jnp.float32),
            jax.ShapeDtypeStruct((_SC_NW, 16), jnp.float32),
        ],
        mesh=mesh,
        scratch_types=[
            pltpu.VMEM((_SC_CHUNK_ROWS, 512), jnp.float32),
            pltpu.VMEM((_SC_CHUNK_ROWS, 512), jnp.float32),
            pltpu.VMEM((1, 16), jnp.float32),
            pltpu.VMEM((1, 16), jnp.float32),
            pltpu.SemaphoreType.DMA,
            pltpu.SemaphoreType.DMA,
        ],
    )(nll)

    loss = pl.pallas_call(
        functools.partial(_stagec_body, n_pix),
        in_specs=[
            pl.BlockSpec(memory_space=pltpu.VMEM),
            pl.BlockSpec(memory_space=pltpu.VMEM),
            pl.BlockSpec(memory_space=pl.ANY),
        ],
        out_specs=pl.BlockSpec(memory_space=pltpu.SMEM),
        out_shape=jax.ShapeDtypeStruct((1, 1), jnp.float32),
        scratch_shapes=[
            pltpu.VMEM((n_rows, 512), jnp.float32),
            pltpu.SemaphoreType.DMA,
        ],
    )(c0p, s0p, nll)
    return loss[0, 0]


# speedup vs baseline: 1.4122x; 1.0122x over previous
import functools
import math

import jax
import jax.numpy as jnp
import numpy as np
from jax import lax
from jax.experimental import pallas as pl
from jax.experimental.pallas import tpu as pltpu
from jax.experimental.pallas import tpu_sc as plsc

_THRESH = 0.7
_N_MIN = 131072
_CT_F32 = np.float32(-math.log(_THRESH))
_CT_KEY = int(np.array(_CT_F32, np.float32).view(np.int32))
_KEY_HI = 0x7F800000

_SC_NC = 2
_SC_NS = 16
_SC_NW = _SC_NC * _SC_NS
_SC_CHUNK_ROWS = 32


def _nll_body(lg_ref, lab_ref, nll_ref):
    lg = lg_ref[0]
    lab = lab_ref[0]
    m = jnp.max(lg, axis=0)
    s = jnp.sum(jnp.exp(lg - m[None]), axis=0)
    cidx = lax.broadcasted_iota(jnp.int32, lg.shape, 0)
    x = jnp.sum(jnp.where(cidx == lab[None], lg, 0.0), axis=0)
    nll_ref[...] = m + jnp.log(s) - x


def _sc_part_body(rows_per_w, nll_hbm, c0_hbm, s0_hbm,
                  buf0, buf1, accc, accs, sem0, sem1):
    wid = lax.axis_index("s") * _SC_NC + lax.axis_index("c")
    base = wid * rows_per_w
    nchunk = rows_per_w // _SC_CHUNK_ROWS
    bufs = (buf0, buf1)
    sems = (sem0, sem1)

    def start(ci):
        src = nll_hbm.at[pl.ds(base + ci * _SC_CHUNK_ROWS, _SC_CHUNK_ROWS)]
        return pltpu.async_copy(src, bufs[ci % 2], sems[ci % 2])

    zf = jnp.zeros((16,), jnp.float32)
    acc = (zf,) * 8
    pending = start(0)
    for ci in range(nchunk):
        pending.wait()
        if ci + 1 < nchunk:
            pending = start(ci + 1)
        buf = bufs[ci % 2]

        def row_step(r, a, buf=buf):
            a = list(a)
            for k in range(32):
                v = buf[r, pl.ds(k * 16, 16)]
                msk = v >= _CT_F32
                a[k % 4] = a[k % 4] + jnp.where(msk, 1.0, 0.0)
                a[4 + k % 4] = a[4 + k % 4] + jnp.where(msk, v, 0.0)
            return tuple(a)

        acc = lax.fori_loop(0, _SC_CHUNK_ROWS, row_step, acc)

    accc[0] = acc[0] + acc[1] + acc[2] + acc[3]
    accs[0] = acc[4] + acc[5] + acc[6] + acc[7]
    pltpu.sync_copy(accc, c0_hbm.at[pl.ds(wid, 1)])
    pltpu.sync_copy(accs, s0_hbm.at[pl.ds(wid, 1)])


def _stagec_body(n_pix, c0p_ref, s0p_ref, nll_any, out_ref, nll_vmem, sem):
    c0 = jnp.sum(c0p_ref[...])
    s0 = jnp.sum(s0p_ref[...])

    @pl.when(c0 > jnp.float32(_N_MIN))
    def _fast():
        out_ref[0, 0] = s0 / c0

    @pl.when(c0 <= jnp.float32(_N_MIN))
    def _slow():
        pltpu.make_async_copy(nll_any, nll_vmem, sem).start()
        pltpu.make_async_copy(nll_any, nll_vmem, sem).wait()
        nll = nll_vmem[...]
        keys = lax.bitcast_convert_type(nll, jnp.int32)
        target = jnp.int32(n_pix - _N_MIN)

        def step(_, carry):
            lo, hi = carry
            mid = lo + (hi - lo) // 2
            cnt = jnp.sum((keys <= mid).astype(jnp.int32))
            return (jnp.where(cnt >= target, lo, mid + 1),
                    jnp.where(cnt >= target, mid, hi))

        lo, hi = lax.fori_loop(0, 31, step, (jnp.int32(0), jnp.int32(_KEY_HI)))
        t_key = jnp.minimum(lo, jnp.int32(_CT_KEY))
        valid = keys >= t_key
        cnt_v = jnp.sum(valid.astype(jnp.int32))
        s = jnp.sum(jnp.where(valid, nll, 0.0))
        out_ref[0, 0] = s / jnp.maximum(cnt_v.astype(jnp.float32), 1.0)


def kernel(logits, labels):
    n, c, h, w = logits.shape
    bh = 512
    n_pix = n * h * w
    n_rows = n_pix // 512
    bpn = h // bh
    nll = pl.pallas_call(
        _nll_body,
        grid=(n, bpn),
        in_specs=[
            pl.BlockSpec((1, c, bh, w), lambda i, j: (i, 0, j, 0)),
            pl.BlockSpec((1, bh, w), lambda i, j: (i, j, 0)),
        ],
        out_specs=pl.BlockSpec((bh, w), lambda i, j, bpn=bpn: (i * bpn + j, 0)),
        out_shape=jax.ShapeDtypeStruct((n_rows, 512), jnp.float32),
    )(logits, labels)

    rows_per_w = n_rows // _SC_NW
    mesh = plsc.VectorSubcoreMesh(core_axis_name="c", subcore_axis_name="s")
    c0p, s0p = pl.kernel(
        functools.partial(_sc_part_body, rows_per_w),
        out_type=[
            jax.ShapeDtypeStruct((_SC_NW, 16), jnp.float32),
            jax.ShapeDtypeStruct((_SC_NW, 16), jnp.float32),
        ],
        mesh=mesh,
        scratch_types=[
            pltpu.VMEM((_SC_CHUNK_ROWS, 512), jnp.float32),
            pltpu.VMEM((_SC_CHUNK_ROWS, 512), jnp.float32),
            pltpu.VMEM((1, 16), jnp.float32),
            pltpu.VMEM((1, 16), jnp.float32),
            pltpu.SemaphoreType.DMA,
            pltpu.SemaphoreType.DMA,
        ],
    )(nll)

    loss = pl.pallas_call(
        functools.partial(_stagec_body, n_pix),
        in_specs=[
            pl.BlockSpec(memory_space=pltpu.VMEM),
            pl.BlockSpec(memory_space=pltpu.VMEM),
            pl.BlockSpec(memory_space=pl.ANY),
        ],
        out_specs=pl.BlockSpec(memory_space=pltpu.SMEM),
        out_shape=jax.ShapeDtypeStruct((1, 1), jnp.float32),
        scratch_shapes=[
            pltpu.VMEM((n_rows, 512), jnp.float32),
            pltpu.SemaphoreType.DMA,
        ],
    )(c0p, s0p, nll)
    return loss[0, 0]
